# baseline (speedup 1.0000x reference)
import jax
import jax.numpy as jnp
from jax import lax
from jax.experimental import pallas as pl
from jax.experimental.pallas import tpu as pltpu
from jax.experimental.pallas import tpu_sc as plsc

N = 10000
HID = 16
NCORE = 2
NSUB = 16
NTILE = NCORE * NSUB
BLK = 128
EPT = 9984
XOFF = NTILE * EPT
CLEN = (2560, 2560, 2560, 2304)
COFF = (0, 2560, 5120, 7680)
CMAX = 2560
NPAD = 10112
RPT = NPAD // NSUB
NV = N // 8
NPV = NPAD // 8

_f32 = jnp.float32


def _mesh():
    return plsc.VectorSubcoreMesh(core_axis_name="c", subcore_axis_name="s")


_SC_PARAMS = pltpu.CompilerParams(use_tc_tiling_on_sc=False)
_HI = lax.Precision.HIGHEST


def _fill_rows(ref, nrows, vec):
    @pl.loop(0, nrows)
    def _(r):
        ref.at[r][...] = vec


def _deg_sc_body(s0, s1, s2, d0, d1, d2, hot_hbm, out_hbm,
                 ia0, ia1, ia2, ia3, ib0, ib1, ib2, ib3, xa, xb,
                 slab_a, slab_b, zbuf, acc, ssem):
    arrs = [s0, s1, s2, d0, d1, d2]
    ibufs = [[ia0, ia1, ia2, ia3], [ib0, ib1, ib2, ib3]]
    xbufs = [xa, xb]
    slabs = [slab_a, slab_b]
    cid = lax.axis_index("c")
    sid = lax.axis_index("s")
    wid = sid * NCORE + cid
    base = wid * EPT
    _fill_rows(zbuf, RPT, jnp.zeros((16,), _f32))
    pltpu.sync_copy(zbuf, acc.at[pl.ds(sid * RPT, RPT)])
    plsc.subcore_barrier()
    puts = {}
    for a in range(6):
        arr, bufs, slab, xb_ = arrs[a], ibufs[a % 2], slabs[a % 2], xbufs[a % 2]
        for p in puts.pop(a - 2, ()):
            p.wait()
        for c in range(4):
            pltpu.sync_copy(arr.at[pl.ds(base + COFF[c], CLEN[c])], bufs[c])
        pltpu.sync_copy(hot_hbm.at[a], slab)
        puts[a] = [pltpu.async_copy(slab.at[pl.ds(0, CLEN[c])],
                                    acc.at[bufs[c]], ssem, add=True)
                   for c in range(4)]

        @pl.when(wid < 4)
        def _(arr=arr, slab=slab, xb_=xb_):
            pltpu.sync_copy(arr.at[pl.ds(XOFF + wid * BLK, BLK)], xb_)
            pltpu.async_copy(slab.at[pl.ds(0, BLK)], acc.at[xb_],
                             ssem, add=True).wait()

    for a in (4, 5):
        for p in puts[a]:
            p.wait()
    plsc.subcore_barrier()
    pltpu.sync_copy(acc.at[pl.ds(sid * RPT, RPT)],
                    out_hbm.at[cid, pl.ds(sid * RPT, RPT)])


def _deg_sc(srcs, dsts, hot):
    return pl.kernel(
        _deg_sc_body,
        mesh=_mesh(),
        out_type=jax.ShapeDtypeStruct((NCORE, NPAD, HID), _f32),
        scratch_types=[pltpu.VMEM((c,), jnp.int32) for c in CLEN] * 2
        + [pltpu.VMEM((BLK,), jnp.int32)] * 2
        + [pltpu.VMEM((CMAX, HID), _f32)] * 2
        + [pltpu.VMEM((RPT, HID), _f32),
           pltpu.VMEM_SHARED((NPAD, HID), _f32),
           pltpu.SemaphoreType.DMA],
        compiler_params=_SC_PARAMS,
    )(*srcs, *dsts, hot)


def _agg_sc_body(t0, t1, t2, s0, s1, s2, d0, d1, d2, out_hbm,
                 sc0, sc1, sc2, sc3, dc0, dc1, dc2, dc3, xs, xd,
                 ra, rb, zbuf, acc, gsem, ssem):
    tabs = [t0, t1, t2]
    srcs = [s0, s1, s2]
    dsts = [d0, d1, d2]
    sbufs = [sc0, sc1, sc2, sc3]
    dbufs = [dc0, dc1, dc2, dc3]
    rows = [ra, rb]
    cid = lax.axis_index("c")
    sid = lax.axis_index("s")
    wid = sid * NCORE + cid
    base = wid * EPT
    _fill_rows(zbuf, RPT, jnp.zeros((16,), _f32))
    pltpu.sync_copy(zbuf, acc.at[pl.ds(sid * RPT, RPT)])
    plsc.subcore_barrier()
    for e in range(3):
        tab = tabs[e]
        for c in range(4):
            pltpu.sync_copy(srcs[e].at[pl.ds(base + COFF[c], CLEN[c])],
                            sbufs[c])
            pltpu.sync_copy(dsts[e].at[pl.ds(base + COFF[c], CLEN[c])],
                            dbufs[c])
        g = [None] * 4
        s = [None] * 4
        g[0] = pltpu.async_copy(tab.at[sbufs[0]], rows[0], gsem)
        for m in range(4):
            rbuf = rows[m % 2].at[pl.ds(0, CLEN[m])]
            g[m].wait()
            if m >= 1:
                s[m - 1].wait()
            if m + 1 < 4:
                g[m + 1] = pltpu.async_copy(
                    tab.at[sbufs[m + 1]],
                    rows[(m + 1) % 2].at[pl.ds(0, CLEN[m + 1])], gsem)
            s[m] = pltpu.async_copy(rbuf, acc.at[dbufs[m]], ssem, add=True)
        s[3].wait()

        @pl.when(wid < 4)
        def _(tab=tab, e=e):
            pltpu.sync_copy(srcs[e].at[pl.ds(XOFF + wid * BLK, BLK)], xs)
            pltpu.sync_copy(dsts[e].at[pl.ds(XOFF + wid * BLK, BLK)], xd)
            pltpu.async_copy(tab.at[xs], rows[0].at[pl.ds(0, BLK)],
                             gsem).wait()
            pltpu.async_copy(rows[0].at[pl.ds(0, BLK)], acc.at[xd],
                             ssem, add=True).wait()

        plsc.subcore_barrier()
        pltpu.sync_copy(acc.at[pl.ds(sid * RPT, RPT)],
                        out_hbm.at[cid, e, pl.ds(sid * RPT, RPT)])
        if e < 2:
            pltpu.sync_copy(zbuf, acc.at[pl.ds(sid * RPT, RPT)])
            plsc.subcore_barrier()


def _agg_sc(tabs, srcs, dsts):
    return pl.kernel(
        _agg_sc_body,
        mesh=_mesh(),
        out_type=jax.ShapeDtypeStruct((NCORE, 3, NPAD, HID), _f32),
        scratch_types=[pltpu.VMEM((c,), jnp.int32) for c in CLEN] * 2
        + [pltpu.VMEM((BLK,), jnp.int32)] * 2
        + [pltpu.VMEM((CMAX, HID), _f32)] * 2
        + [pltpu.VMEM((RPT, HID), _f32),
           pltpu.VMEM_SHARED((NPAD, HID), _f32),
           pltpu.SemaphoreType.DMA, pltpu.SemaphoreType.DMA],
        compiler_params=_SC_PARAMS,
    )(*tabs, *srcs, *dsts)


def _mm_body(x_ref, w_ref, o_ref):
    o_ref[...] = jnp.dot(x_ref[...], w_ref[...],
                         preferred_element_type=_f32, precision=_HI)


def _h1_tc(x, w):
    return pl.pallas_call(
        _mm_body,
        grid=(10,),
        in_specs=[pl.BlockSpec((1000, 128), lambda i: (i, 0)),
                  pl.BlockSpec((128, 48), lambda i: (0, 0))],
        out_specs=pl.BlockSpec((1000, 48), lambda i: (i, 0)),
        out_shape=jax.ShapeDtypeStruct((N, 48), _f32),
    )(x, w)


def _rsqrt_deg(d_ref):
    d = d_ref[0] + d_ref[1]
    return lax.rsqrt(jnp.maximum(d, 1.0))[:NV]


def _tab_body(h_ref, d_ref, s_ref, m_ref, o0, o1, o2):
    r = _rsqrt_deg(d_ref)
    h = h_ref[...]
    outs = [o0, o1, o2]
    for e in range(3):
        sc = jnp.dot(r, m_ref[e], preferred_element_type=_f32, precision=_HI)
        t = jnp.dot(h, s_ref[e], preferred_element_type=_f32, precision=_HI)
        outs[e][...] = t * sc


def _tab_tc(h1v, degv, sel, ms):
    full = lambda shape: pl.BlockSpec(shape, lambda: tuple(0 for _ in shape))
    return pl.pallas_call(
        _tab_body,
        in_specs=[full((NV, 384)), full((NCORE, NPV, BLK)),
                  full((3, 384, BLK)), full((6, BLK, BLK))],
        out_specs=[full((NV, BLK))] * 3,
        out_shape=[jax.ShapeDtypeStruct((NV, BLK), _f32)] * 3,
    )(h1v, degv, sel, ms)


def _mid_body(a_ref, d_ref, b_ref, m_ref, o0, o1, o2):
    r = _rsqrt_deg(d_ref)
    h = jnp.broadcast_to(b_ref[0:1, :], (NV, BLK))
    for e in range(3):
        insc = jnp.dot(r, m_ref[3 + e], preferred_element_type=_f32,
                       precision=_HI)
        h = h + (a_ref[0, e] + a_ref[1, e])[:NV] * insc
    h = jnp.maximum(h, 0.0)
    outs = [o0, o1, o2]
    for e in range(3):
        outsc = jnp.dot(r, m_ref[e], preferred_element_type=_f32,
                        precision=_HI)
        outs[e][...] = h * outsc


def _mid_tc(aggv, degv, b1t, ms):
    full = lambda shape: pl.BlockSpec(shape, lambda: tuple(0 for _ in shape))
    return pl.pallas_call(
        _mid_body,
        in_specs=[full((NCORE, 3, NPV, BLK)), full((NCORE, NPV, BLK)),
                  full((8, BLK)), full((6, BLK, BLK))],
        out_specs=[full((NV, BLK))] * 3,
        out_shape=[jax.ShapeDtypeStruct((NV, BLK), _f32)] * 3,
    )(aggv, degv, b1t, ms)


def _out_body(a_ref, d_ref, w_ref, b_ref, m_ref, o_ref):
    r = _rsqrt_deg(d_ref)
    acc = jnp.broadcast_to(b_ref[0:1, :], (NV, HID))
    for e in range(3):
        insc = jnp.dot(r, m_ref[3 + e], preferred_element_type=_f32,
                       precision=_HI)
        m = (a_ref[0, e] + a_ref[1, e])[:NV] * insc
        acc = acc + jnp.dot(m, w_ref[e], preferred_element_type=_f32,
                            precision=_HI)
    o_ref[...] = acc


def _out_tc(aggv, degv, w2b, b2t, ms):
    full = lambda shape: pl.BlockSpec(shape, lambda: tuple(0 for _ in shape))
    return pl.pallas_call(
        _out_body,
        in_specs=[full((NCORE, 3, NPV, BLK)), full((NCORE, NPV, BLK)),
                  full((3, BLK, HID)), full((8, HID)), full((6, BLK, BLK))],
        out_specs=full((NV, HID)),
        out_shape=jax.ShapeDtypeStruct((NV, HID), _f32),
    )(aggv, degv, w2b, b2t, ms)


def kernel(x, edge_index_rsr, edge_index_rtr, edge_index_rur,
           W1_rsr, b1_rsr, W1_rtr, b1_rtr, W1_rur, b1_rur,
           W2_rsr, b2_rsr, W2_rtr, b2_rtr, W2_rur, b2_rur):
    eis = [edge_index_rsr, edge_index_rtr, edge_index_rur]
    srcs = [ei[0] for ei in eis]
    dsts = [ei[1] for ei in eis]

    eye8 = jnp.eye(8, dtype=_f32)
    eye48 = jnp.eye(48, dtype=_f32)
    ones16 = jnp.ones((16,), _f32)
    sel = jnp.stack([jnp.kron(eye8, eye48[:, 16 * e:16 * (e + 1)])
                     for e in range(3)])
    ms = jnp.stack([jnp.kron(eye8, jnp.outer(jnp.eye(16, dtype=_f32)[a],
                                             ones16))
                    for a in range(6)])
    hot = jnp.stack([jnp.tile(jnp.where(jnp.arange(16) == a, 1.0, 0.0)
                              .astype(_f32)[None, :], (CMAX, 1))
                     for a in range(6)])
    w2b = jnp.stack([jnp.kron(eye8, w) for w in (W2_rsr, W2_rtr, W2_rur)])
    b1t = jnp.broadcast_to(jnp.tile(b1_rsr + b1_rtr + b1_rur, 8), (8, BLK))
    b2t = jnp.broadcast_to(jnp.tile(b2_rsr + b2_rtr + b2_rur, 8), (8, HID))
    w1 = jnp.concatenate([W1_rsr, W1_rtr, W1_rur], axis=1)

    degs = _deg_sc(srcs, dsts, hot)
    degv = degs.reshape(NCORE, NPV, BLK)
    h1 = _h1_tc(x, w1)
    h1v = h1.reshape(NV, 384)

    t1 = _tab_tc(h1v, degv, sel, ms)
    tabs1 = [t.reshape(N, HID) for t in t1]
    a1 = _agg_sc(tabs1, srcs, dsts)
    a1v = a1.reshape(NCORE, 3, NPV, BLK)

    t2 = _mid_tc(a1v, degv, b1t, ms)
    tabs2 = [t.reshape(N, HID) for t in t2]
    a2 = _agg_sc(tabs2, srcs, dsts)
    a2v = a2.reshape(NCORE, 3, NPV, BLK)

    out = _out_tc(a2v, degv, w2b, b2t, ms)
    return out.reshape(N, 2)


# speedup vs baseline: 1.2766x; 1.2766x over previous
import jax
import jax.numpy as jnp
from jax import lax
from jax.experimental import pallas as pl
from jax.experimental.pallas import tpu as pltpu
from jax.experimental.pallas import tpu_sc as plsc

N = 10000
HID = 16
NCORE = 2
NSUB = 16
NTILE = NCORE * NSUB
BLK = 128
NROW = 2500
TPB = 78
XBASE = NTILE * TPB
KB = 26
NBAT = TPB // KB
NPAD = 10112
RPT = NPAD // NSUB
NV = N // 8
NPV = NPAD // 8

_f32 = jnp.float32


def _mesh():
    return plsc.VectorSubcoreMesh(core_axis_name="c", subcore_axis_name="s")


_SC_PARAMS = pltpu.CompilerParams(use_tc_tiling_on_sc=False)
_HI = lax.Precision.HIGHEST


def _fill_rows(ref, nrows, vec):
    @pl.loop(0, nrows)
    def _(r):
        ref.at[r][...] = vec


def _deg_sc_body(e0, e1, e2, out_hbm, i0, i1, i2, i3, i4, i5, x6,
                 h0, h1, h2, h3, h4, h5, zbuf, acc, ssem):
    eis = [e0, e1, e2]
    idxs = [i0, i1, i2, i3, i4, i5]
    hots = [h0, h1, h2, h3, h4, h5]
    cid = lax.axis_index("c")
    sid = lax.axis_index("s")
    wid = sid * NCORE + cid
    _fill_rows(zbuf, RPT, jnp.zeros((16,), _f32))
    pltpu.sync_copy(zbuf, acc.at[pl.ds(sid * RPT, RPT)])
    for a in range(6):
        onehot = jnp.where(lax.iota(jnp.int32, 16) == a, 1.0, 0.0).astype(_f32)
        _fill_rows(hots[a], BLK, onehot)
        pltpu.sync_copy(eis[a % 3].at[a // 3, pl.ds(wid * TPB, TPB)], idxs[a])

    @pl.when(wid < 4)
    def _():
        for a in range(6):
            pltpu.sync_copy(eis[a % 3].at[a // 3, pl.ds(XBASE + wid, 1)],
                            x6.at[pl.ds(a, 1)])

    plsc.subcore_barrier()
    for a in range(6):

        @pl.loop(0, TPB)
        def _(j, hot=hots[a], idx=idxs[a]):
            pltpu.async_copy(hot, acc.at[idx.at[j]], ssem, add=True)

    @pl.when(wid < 4)
    def _():
        for a in range(6):
            pltpu.async_copy(hots[a], acc.at[x6.at[a]], ssem, add=True)

    @pl.loop(0, 6 * TPB)
    def _(j):
        pltpu.make_async_copy(out_hbm.at[cid, pl.ds(0, BLK)], h0, ssem).wait()

    @pl.when(wid < 4)
    def _():
        for a in range(6):
            pltpu.make_async_copy(out_hbm.at[cid, pl.ds(0, BLK)], h0,
                                  ssem).wait()

    plsc.subcore_barrier()
    pltpu.sync_copy(acc.at[pl.ds(sid * RPT, RPT)],
                    out_hbm.at[cid, pl.ds(sid * RPT, RPT)])


def _deg_sc(e0, e1, e2):
    return pl.kernel(
        _deg_sc_body,
        mesh=_mesh(),
        out_type=jax.ShapeDtypeStruct((NCORE, NPAD, HID), _f32),
        scratch_types=[pltpu.VMEM((TPB, BLK), jnp.int32)] * 6
        + [pltpu.VMEM((6, BLK), jnp.int32)]
        + [pltpu.VMEM((BLK, HID), _f32)] * 6
        + [pltpu.VMEM((RPT, HID), _f32),
           pltpu.VMEM_SHARED((NPAD, HID), _f32),
           pltpu.SemaphoreType.DMA],
        compiler_params=_SC_PARAMS,
    )(e0, e1, e2)


def _agg_sc_body(t0, t1, t2, e0, e1, e2, out_hbm, src_v, dst_v, x2,
                 *rest):
    rows = list(rest[:KB])
    zbuf, acc, gsem, ssem = rest[KB:]
    tabs = [t0, t1, t2]
    eis = [e0, e1, e2]
    cid = lax.axis_index("c")
    sid = lax.axis_index("s")
    wid = sid * NCORE + cid
    _fill_rows(zbuf, RPT, jnp.zeros((16,), _f32))
    pltpu.sync_copy(zbuf, acc.at[pl.ds(sid * RPT, RPT)])
    plsc.subcore_barrier()
    for e in range(3):
        tab, ei = tabs[e], eis[e]
        pltpu.sync_copy(ei.at[0, pl.ds(wid * TPB, TPB)], src_v)
        pltpu.sync_copy(ei.at[1, pl.ds(wid * TPB, TPB)], dst_v)

        @pl.when(wid < 4)
        def _():
            pltpu.sync_copy(ei.at[0, pl.ds(XBASE + wid, 1)], x2.at[pl.ds(0, 1)])
            pltpu.sync_copy(ei.at[1, pl.ds(XBASE + wid, 1)], x2.at[pl.ds(1, 1)])

        @pl.loop(0, NBAT)
        def _(b, tab=tab):
            base = b * KB
            gets = [pltpu.async_copy(tab.at[src_v.at[base + i]], rows[i], gsem)
                    for i in range(KB)]
            puts = []
            for i in range(KB):
                gets[i].wait()
                puts.append(pltpu.async_copy(rows[i],
                                             acc.at[dst_v.at[base + i]],
                                             ssem, add=True))
            for p in puts:
                p.wait()

        @pl.when(wid < 4)
        def _(tab=tab):
            pltpu.async_copy(tab.at[x2.at[0]], rows[0], gsem).wait()
            pltpu.async_copy(rows[0], acc.at[x2.at[1]], ssem, add=True).wait()

        plsc.subcore_barrier()
        pltpu.sync_copy(acc.at[pl.ds(sid * RPT, RPT)],
                        out_hbm.at[cid, e, pl.ds(sid * RPT, RPT)])
        if e < 2:
            pltpu.sync_copy(zbuf, acc.at[pl.ds(sid * RPT, RPT)])
            plsc.subcore_barrier()


def _agg_sc(t0, t1, t2, e0, e1, e2):
    return pl.kernel(
        _agg_sc_body,
        mesh=_mesh(),
        out_type=jax.ShapeDtypeStruct((NCORE, 3, NPAD, HID), _f32),
        scratch_types=[pltpu.VMEM((TPB, BLK), jnp.int32),
                       pltpu.VMEM((TPB, BLK), jnp.int32),
                       pltpu.VMEM((2, BLK), jnp.int32)]
        + [pltpu.VMEM((BLK, HID), _f32)] * KB
        + [pltpu.VMEM((RPT, HID), _f32),
           pltpu.VMEM_SHARED((NPAD, HID), _f32),
           pltpu.SemaphoreType.DMA, pltpu.SemaphoreType.DMA],
        compiler_params=_SC_PARAMS,
    )(t0, t1, t2, e0, e1, e2)


def _mm_body(x_ref, w_ref, o_ref):
    o_ref[...] = jnp.dot(x_ref[...], w_ref[...],
                         preferred_element_type=_f32, precision=_HI)


def _h1_tc(x, w):
    return pl.pallas_call(
        _mm_body,
        grid=(10,),
        in_specs=[pl.BlockSpec((1000, 128), lambda i: (i, 0)),
                  pl.BlockSpec((128, 48), lambda i: (0, 0))],
        out_specs=pl.BlockSpec((1000, 48), lambda i: (i, 0)),
        out_shape=jax.ShapeDtypeStruct((N, 48), _f32),
    )(x, w)


def _rsqrt_deg(d_ref):
    d = d_ref[0] + d_ref[1]
    return lax.rsqrt(jnp.maximum(d, 1.0))[:NV]


def _tab_body(h_ref, d_ref, s_ref, m_ref, o0, o1, o2):
    r = _rsqrt_deg(d_ref)
    h = h_ref[...]
    outs = [o0, o1, o2]
    for e in range(3):
        sc = jnp.dot(r, m_ref[e], preferred_element_type=_f32, precision=_HI)
        t = jnp.dot(h, s_ref[e], preferred_element_type=_f32, precision=_HI)
        outs[e][...] = t * sc


def _tab_tc(h1v, degv, sel, ms):
    full = lambda shape: pl.BlockSpec(shape, lambda: tuple(0 for _ in shape))
    return pl.pallas_call(
        _tab_body,
        in_specs=[full((NV, 384)), full((NCORE, NPV, BLK)),
                  full((3, 384, BLK)), full((6, BLK, BLK))],
        out_specs=[full((NV, BLK))] * 3,
        out_shape=[jax.ShapeDtypeStruct((NV, BLK), _f32)] * 3,
    )(h1v, degv, sel, ms)


def _mid_body(a_ref, d_ref, b_ref, m_ref, o0, o1, o2):
    r = _rsqrt_deg(d_ref)
    h = jnp.broadcast_to(b_ref[0:1, :], (NV, BLK))
    for e in range(3):
        insc = jnp.dot(r, m_ref[3 + e], preferred_element_type=_f32,
                       precision=_HI)
        h = h + (a_ref[0, e] + a_ref[1, e])[:NV] * insc
    h = jnp.maximum(h, 0.0)
    outs = [o0, o1, o2]
    for e in range(3):
        outsc = jnp.dot(r, m_ref[e], preferred_element_type=_f32,
                        precision=_HI)
        outs[e][...] = h * outsc


def _mid_tc(aggv, degv, b1t, ms):
    full = lambda shape: pl.BlockSpec(shape, lambda: tuple(0 for _ in shape))
    return pl.pallas_call(
        _mid_body,
        in_specs=[full((NCORE, 3, NPV, BLK)), full((NCORE, NPV, BLK)),
                  full((8, BLK)), full((6, BLK, BLK))],
        out_specs=[full((NV, BLK))] * 3,
        out_shape=[jax.ShapeDtypeStruct((NV, BLK), _f32)] * 3,
    )(aggv, degv, b1t, ms)


def _out_body(a_ref, d_ref, w_ref, b_ref, m_ref, o_ref):
    r = _rsqrt_deg(d_ref)
    acc = jnp.broadcast_to(b_ref[0:1, :], (NV, HID))
    for e in range(3):
        insc = jnp.dot(r, m_ref[3 + e], preferred_element_type=_f32,
                       precision=_HI)
        m = (a_ref[0, e] + a_ref[1, e])[:NV] * insc
        acc = acc + jnp.dot(m, w_ref[e], preferred_element_type=_f32,
                            precision=_HI)
    o_ref[...] = acc


def _out_tc(aggv, degv, w2b, b2t, ms):
    full = lambda shape: pl.BlockSpec(shape, lambda: tuple(0 for _ in shape))
    return pl.pallas_call(
        _out_body,
        in_specs=[full((NCORE, 3, NPV, BLK)), full((NCORE, NPV, BLK)),
                  full((3, BLK, HID)), full((8, HID)), full((6, BLK, BLK))],
        out_specs=full((NV, HID)),
        out_shape=jax.ShapeDtypeStruct((NV, HID), _f32),
    )(aggv, degv, w2b, b2t, ms)


def kernel(x, edge_index_rsr, edge_index_rtr, edge_index_rur,
           W1_rsr, b1_rsr, W1_rtr, b1_rtr, W1_rur, b1_rur,
           W2_rsr, b2_rsr, W2_rtr, b2_rtr, W2_rur, b2_rur):
    e0 = edge_index_rsr.reshape(2, NROW, BLK)
    e1 = edge_index_rtr.reshape(2, NROW, BLK)
    e2 = edge_index_rur.reshape(2, NROW, BLK)

    eye8 = jnp.eye(8, dtype=_f32)
    eye48 = jnp.eye(48, dtype=_f32)
    ones16 = jnp.ones((16,), _f32)
    sel = jnp.stack([jnp.kron(eye8, eye48[:, 16 * e:16 * (e + 1)])
                     for e in range(3)])
    ms = jnp.stack([jnp.kron(eye8, jnp.outer(jnp.eye(16, dtype=_f32)[a],
                                             ones16))
                    for a in range(6)])
    w2b = jnp.stack([jnp.kron(eye8, w) for w in (W2_rsr, W2_rtr, W2_rur)])
    b1t = jnp.broadcast_to(jnp.tile(b1_rsr + b1_rtr + b1_rur, 8), (8, BLK))
    b2t = jnp.broadcast_to(jnp.tile(b2_rsr + b2_rtr + b2_rur, 8), (8, HID))
    w1 = jnp.concatenate([W1_rsr, W1_rtr, W1_rur], axis=1)

    degs = _deg_sc(e0, e1, e2)
    degv = degs.reshape(NCORE, NPV, BLK)
    h1 = _h1_tc(x, w1)
    h1v = h1.reshape(NV, 384)

    t1 = _tab_tc(h1v, degv, sel, ms)
    tabs1 = [t.reshape(N, HID) for t in t1]
    a1 = _agg_sc(*tabs1, e0, e1, e2)
    a1v = a1.reshape(NCORE, 3, NPV, BLK)

    t2 = _mid_tc(a1v, degv, b1t, ms)
    tabs2 = [t.reshape(N, HID) for t in t2]
    a2 = _agg_sc(*tabs2, e0, e1, e2)
    a2v = a2.reshape(NCORE, 3, NPV, BLK)

    out = _out_tc(a2v, degv, w2b, b2t, ms)
    return out.reshape(N, 2)


# speedup vs baseline: 1.2818x; 1.0041x over previous
import jax
import jax.numpy as jnp
from jax import lax
from jax.experimental import pallas as pl
from jax.experimental.pallas import tpu as pltpu
from jax.experimental.pallas import tpu_sc as plsc

N = 10000
HID = 16
NCORE = 2
NSUB = 16
NTILE = NCORE * NSUB
BLK = 128
NROW = 2500
TPB = 78
XBASE = NTILE * TPB
KB = 39
NBAT = TPB // KB
NPAD = 10112
RPT = NPAD // NSUB
NV = N // 8
NPV = NPAD // 8

_f32 = jnp.float32


def _mesh():
    return plsc.VectorSubcoreMesh(core_axis_name="c", subcore_axis_name="s")


_SC_PARAMS = pltpu.CompilerParams(use_tc_tiling_on_sc=False)
_HI = lax.Precision.HIGHEST


def _fill_rows(ref, nrows, vec):
    @pl.loop(0, nrows)
    def _(r):
        ref.at[r][...] = vec


def _deg_sc_body(e0, e1, e2, out_hbm, i0, i1, i2, i3, i4, i5, x6,
                 h0, h1, h2, h3, h4, h5, zbuf, acc, ssem):
    eis = [e0, e1, e2]
    idxs = [i0, i1, i2, i3, i4, i5]
    hots = [h0, h1, h2, h3, h4, h5]
    cid = lax.axis_index("c")
    sid = lax.axis_index("s")
    wid = sid * NCORE + cid
    _fill_rows(zbuf, RPT, jnp.zeros((16,), _f32))
    pltpu.sync_copy(zbuf, acc.at[pl.ds(sid * RPT, RPT)])
    for a in range(6):
        onehot = jnp.where(lax.iota(jnp.int32, 16) == a, 1.0, 0.0).astype(_f32)
        _fill_rows(hots[a], BLK, onehot)
        pltpu.sync_copy(eis[a % 3].at[a // 3, pl.ds(wid * TPB, TPB)], idxs[a])

    @pl.when(wid < 4)
    def _():
        for a in range(6):
            pltpu.sync_copy(eis[a % 3].at[a // 3, pl.ds(XBASE + wid, 1)],
                            x6.at[pl.ds(a, 1)])

    plsc.subcore_barrier()
    for a in range(6):

        @pl.loop(0, TPB)
        def _(j, hot=hots[a], idx=idxs[a]):
            pltpu.async_copy(hot, acc.at[idx.at[j]], ssem, add=True)

    @pl.when(wid < 4)
    def _():
        for a in range(6):
            pltpu.async_copy(hots[a], acc.at[x6.at[a]], ssem, add=True)

    @pl.loop(0, 6 * TPB)
    def _(j):
        pltpu.make_async_copy(out_hbm.at[cid, pl.ds(0, BLK)], h0, ssem).wait()

    @pl.when(wid < 4)
    def _():
        for a in range(6):
            pltpu.make_async_copy(out_hbm.at[cid, pl.ds(0, BLK)], h0,
                                  ssem).wait()

    plsc.subcore_barrier()
    pltpu.sync_copy(acc.at[pl.ds(sid * RPT, RPT)],
                    out_hbm.at[cid, pl.ds(sid * RPT, RPT)])


def _deg_sc(e0, e1, e2):
    return pl.kernel(
        _deg_sc_body,
        mesh=_mesh(),
        out_type=jax.ShapeDtypeStruct((NCORE, NPAD, HID), _f32),
        scratch_types=[pltpu.VMEM((TPB, BLK), jnp.int32)] * 6
        + [pltpu.VMEM((6, BLK), jnp.int32)]
        + [pltpu.VMEM((BLK, HID), _f32)] * 6
        + [pltpu.VMEM((RPT, HID), _f32),
           pltpu.VMEM_SHARED((NPAD, HID), _f32),
           pltpu.SemaphoreType.DMA],
        compiler_params=_SC_PARAMS,
    )(e0, e1, e2)


def _agg_sc_body(t0, t1, t2, e0, e1, e2, out_hbm, src_v, dst_v, x2,
                 *rest):
    rows = list(rest[:KB])
    zbuf, acc, gsem, ssem = rest[KB:]
    tabs = [t0, t1, t2]
    eis = [e0, e1, e2]
    cid = lax.axis_index("c")
    sid = lax.axis_index("s")
    wid = sid * NCORE + cid
    _fill_rows(zbuf, RPT, jnp.zeros((16,), _f32))
    pltpu.sync_copy(zbuf, acc.at[pl.ds(sid * RPT, RPT)])
    plsc.subcore_barrier()
    for e in range(3):
        tab, ei = tabs[e], eis[e]
        pltpu.sync_copy(ei.at[0, pl.ds(wid * TPB, TPB)], src_v)
        pltpu.sync_copy(ei.at[1, pl.ds(wid * TPB, TPB)], dst_v)

        @pl.when(wid < 4)
        def _():
            pltpu.sync_copy(ei.at[0, pl.ds(XBASE + wid, 1)], x2.at[pl.ds(0, 1)])
            pltpu.sync_copy(ei.at[1, pl.ds(XBASE + wid, 1)], x2.at[pl.ds(1, 1)])

        @pl.loop(0, NBAT)
        def _(b, tab=tab):
            base = b * KB
            gets = [pltpu.async_copy(tab.at[src_v.at[base + i]], rows[i], gsem)
                    for i in range(KB)]
            puts = []
            for i in range(KB):
                gets[i].wait()
                puts.append(pltpu.async_copy(rows[i],
                                             acc.at[dst_v.at[base + i]],
                                             ssem, add=True))
            for p in puts:
                p.wait()

        @pl.when(wid < 4)
        def _(tab=tab):
            pltpu.async_copy(tab.at[x2.at[0]], rows[0], gsem).wait()
            pltpu.async_copy(rows[0], acc.at[x2.at[1]], ssem, add=True).wait()

        plsc.subcore_barrier()
        pltpu.sync_copy(acc.at[pl.ds(sid * RPT, RPT)],
                        out_hbm.at[cid, e, pl.ds(sid * RPT, RPT)])
        if e < 2:
            pltpu.sync_copy(zbuf, acc.at[pl.ds(sid * RPT, RPT)])
            plsc.subcore_barrier()


def _agg_sc(t0, t1, t2, e0, e1, e2):
    return pl.kernel(
        _agg_sc_body,
        mesh=_mesh(),
        out_type=jax.ShapeDtypeStruct((NCORE, 3, NPAD, HID), _f32),
        scratch_types=[pltpu.VMEM((TPB, BLK), jnp.int32),
                       pltpu.VMEM((TPB, BLK), jnp.int32),
                       pltpu.VMEM((2, BLK), jnp.int32)]
        + [pltpu.VMEM((BLK, HID), _f32)] * KB
        + [pltpu.VMEM((RPT, HID), _f32),
           pltpu.VMEM_SHARED((NPAD, HID), _f32),
           pltpu.SemaphoreType.DMA, pltpu.SemaphoreType.DMA],
        compiler_params=_SC_PARAMS,
    )(t0, t1, t2, e0, e1, e2)


def _mm_body(x_ref, w_ref, o_ref):
    o_ref[...] = jnp.dot(x_ref[...], w_ref[...],
                         preferred_element_type=_f32, precision=_HI)


def _h1_tc(x, w):
    return pl.pallas_call(
        _mm_body,
        grid=(10,),
        in_specs=[pl.BlockSpec((1000, 128), lambda i: (i, 0)),
                  pl.BlockSpec((128, 48), lambda i: (0, 0))],
        out_specs=pl.BlockSpec((1000, 48), lambda i: (i, 0)),
        out_shape=jax.ShapeDtypeStruct((N, 48), _f32),
    )(x, w)


def _rsqrt_deg(d_ref):
    d = d_ref[0] + d_ref[1]
    return lax.rsqrt(jnp.maximum(d, 1.0))[:NV]


def _tab_body(h_ref, d_ref, s_ref, m_ref, o0, o1, o2):
    r = _rsqrt_deg(d_ref)
    h = h_ref[...]
    outs = [o0, o1, o2]
    for e in range(3):
        sc = jnp.dot(r, m_ref[e], preferred_element_type=_f32, precision=_HI)
        t = jnp.dot(h, s_ref[e], preferred_element_type=_f32, precision=_HI)
        outs[e][...] = t * sc


def _tab_tc(h1v, degv, sel, ms):
    full = lambda shape: pl.BlockSpec(shape, lambda: tuple(0 for _ in shape))
    return pl.pallas_call(
        _tab_body,
        in_specs=[full((NV, 384)), full((NCORE, NPV, BLK)),
                  full((3, 384, BLK)), full((6, BLK, BLK))],
        out_specs=[full((NV, BLK))] * 3,
        out_shape=[jax.ShapeDtypeStruct((NV, BLK), _f32)] * 3,
    )(h1v, degv, sel, ms)


def _mid_body(a_ref, d_ref, b_ref, m_ref, o0, o1, o2):
    r = _rsqrt_deg(d_ref)
    h = jnp.broadcast_to(b_ref[0:1, :], (NV, BLK))
    for e in range(3):
        insc = jnp.dot(r, m_ref[3 + e], preferred_element_type=_f32,
                       precision=_HI)
        h = h + (a_ref[0, e] + a_ref[1, e])[:NV] * insc
    h = jnp.maximum(h, 0.0)
    outs = [o0, o1, o2]
    for e in range(3):
        outsc = jnp.dot(r, m_ref[e], preferred_element_type=_f32,
                        precision=_HI)
        outs[e][...] = h * outsc


def _mid_tc(aggv, degv, b1t, ms):
    full = lambda shape: pl.BlockSpec(shape, lambda: tuple(0 for _ in shape))
    return pl.pallas_call(
        _mid_body,
        in_specs=[full((NCORE, 3, NPV, BLK)), full((NCORE, NPV, BLK)),
                  full((8, BLK)), full((6, BLK, BLK))],
        out_specs=[full((NV, BLK))] * 3,
        out_shape=[jax.ShapeDtypeStruct((NV, BLK), _f32)] * 3,
    )(aggv, degv, b1t, ms)


def _out_body(a_ref, d_ref, w_ref, b_ref, m_ref, o_ref):
    r = _rsqrt_deg(d_ref)
    acc = jnp.broadcast_to(b_ref[0:1, :], (NV, HID))
    for e in range(3):
        insc = jnp.dot(r, m_ref[3 + e], preferred_element_type=_f32,
                       precision=_HI)
        m = (a_ref[0, e] + a_ref[1, e])[:NV] * insc
        acc = acc + jnp.dot(m, w_ref[e], preferred_element_type=_f32,
                            precision=_HI)
    o_ref[...] = acc


def _out_tc(aggv, degv, w2b, b2t, ms):
    full = lambda shape: pl.BlockSpec(shape, lambda: tuple(0 for _ in shape))
    return pl.pallas_call(
        _out_body,
        in_specs=[full((NCORE, 3, NPV, BLK)), full((NCORE, NPV, BLK)),
                  full((3, BLK, HID)), full((8, HID)), full((6, BLK, BLK))],
        out_specs=full((NV, HID)),
        out_shape=jax.ShapeDtypeStruct((NV, HID), _f32),
    )(aggv, degv, w2b, b2t, ms)


def kernel(x, edge_index_rsr, edge_index_rtr, edge_index_rur,
           W1_rsr, b1_rsr, W1_rtr, b1_rtr, W1_rur, b1_rur,
           W2_rsr, b2_rsr, W2_rtr, b2_rtr, W2_rur, b2_rur):
    e0 = edge_index_rsr.reshape(2, NROW, BLK)
    e1 = edge_index_rtr.reshape(2, NROW, BLK)
    e2 = edge_index_rur.reshape(2, NROW, BLK)

    eye8 = jnp.eye(8, dtype=_f32)
    eye48 = jnp.eye(48, dtype=_f32)
    ones16 = jnp.ones((16,), _f32)
    sel = jnp.stack([jnp.kron(eye8, eye48[:, 16 * e:16 * (e + 1)])
                     for e in range(3)])
    ms = jnp.stack([jnp.kron(eye8, jnp.outer(jnp.eye(16, dtype=_f32)[a],
                                             ones16))
                    for a in range(6)])
    w2b = jnp.stack([jnp.kron(eye8, w) for w in (W2_rsr, W2_rtr, W2_rur)])
    b1t = jnp.broadcast_to(jnp.tile(b1_rsr + b1_rtr + b1_rur, 8), (8, BLK))
    b2t = jnp.broadcast_to(jnp.tile(b2_rsr + b2_rtr + b2_rur, 8), (8, HID))
    w1 = jnp.concatenate([W1_rsr, W1_rtr, W1_rur], axis=1)

    degs = _deg_sc(e0, e1, e2)
    degv = degs.reshape(NCORE, NPV, BLK)
    h1 = _h1_tc(x, w1)
    h1v = h1.reshape(NV, 384)

    t1 = _tab_tc(h1v, degv, sel, ms)
    tabs1 = [t.reshape(N, HID) for t in t1]
    a1 = _agg_sc(*tabs1, e0, e1, e2)
    a1v = a1.reshape(NCORE, 3, NPV, BLK)

    t2 = _mid_tc(a1v, degv, b1t, ms)
    tabs2 = [t.reshape(N, HID) for t in t2]
    a2 = _agg_sc(*tabs2, e0, e1, e2)
    a2v = a2.reshape(NCORE, 3, NPV, BLK)

    out = _out_tc(a2v, degv, w2b, b2t, ms)
    return out.reshape(N, 2)
